# baseline (speedup 1.0000x reference)
import functools

import jax
import jax.numpy as jnp
from jax import lax
from jax.experimental import pallas as pl
from jax.experimental.pallas import tpu as pltpu
from jax.experimental.pallas import tpu_sc as plsc

_VOCAB = 1000000
_EMB = 64
_BATCH = 16384
_C = 3

_NC = 2
_NS = 16
_NW = _NC * _NS
_BPW = _BATCH // _NW
_CB = 128
_NCHUNK = _BPW // _CB
_SCALE = float(2 ** 17)
_INV_SCALE2 = float(2 ** -34)


def _lane_perm(v, idx):
    return lax.gather(
        v, idx.reshape(16, 1),
        lax.GatherDimensionNumbers(
            offset_dims=(), collapsed_slice_dims=(0,), start_index_map=(0,)),
        slice_sizes=(1,),
        mode=lax.GatherScatterMode.PROMISE_IN_BOUNDS)


def _row_f32(tiles, slot, qoff):
    out = []
    for k in range(2):
        v = tiles[slot, pl.ds(qoff * 32 + 16 * k, 16)]
        lo = (v << 16) >> 16
        hi = v >> 16
        out.append(lo.astype(jnp.float32))
        out.append(hi.astype(jnp.float32))
    return out


def _asm2vec_body(tidx_hbm, cidx_hbm, ttab_hbm, ctab_hbm, out_hbm,
                  tidx_v, cidx_v, tg_v, cg_v, wtiles, ctiles, out_v, sem):
    wid = lax.axis_index("s") * _NC + lax.axis_index("c")
    base = wid * _BPW

    pltpu.sync_copy(tidx_hbm.at[pl.ds(base, _BPW)], tidx_v)
    pltpu.sync_copy(cidx_hbm.at[pl.ds(base * _C, _BPW * _C)], cidx_v)

    def grp_t(j, carry):
        tg_v[pl.ds(16 * j, 16)] = tidx_v[pl.ds(16 * j, 16)] >> 2
        return carry

    def grp_c(j, carry):
        cg_v[pl.ds(16 * j, 16)] = cidx_v[pl.ds(16 * j, 16)] >> 2
        return carry

    lax.fori_loop(0, _BPW // 16, grp_t, 0)
    lax.fori_loop(0, _BPW * _C // 16, grp_c, 0)

    lane = lax.iota(jnp.int32, 16)
    masks = [(lane & sh) != 0 for sh in (1, 2, 4, 8)]
    pidxs = [lane ^ sh for sh in (1, 2, 4, 8)]

    for ck in range(_NCHUNK):
        cb = ck * _CB

        h = [pltpu.async_copy(
            ttab_hbm.at[tg_v.at[pl.ds(cb + 128 * j, 128)]],
            wtiles.at[pl.ds(128 * j, 128)], sem)
            for j in range(_CB // 128)]
        for j in range(_CB * _C // 128):
            h.append(pltpu.async_copy(
                ctab_hbm.at[cg_v.at[pl.ds(cb * _C + 128 * j, 128)]],
                ctiles.at[pl.ds(128 * j, 128)], sem))
        for hh in h:
            hh.wait()

        def body(g, carry):
            b0 = g * 16
            tiv = tidx_v[pl.ds(cb + b0, 16)]
            civ = [cidx_v[pl.ds((cb + b0) * _C + 16 * j, 16)]
                   for j in range(_C)]
            for m in range(_C):
                wcache = {}
                prods = []
                for l in range(16):
                    q = m * 16 + l
                    boff, c = q // _C, q % _C
                    if boff not in wcache:
                        wcache[boff] = _row_f32(wtiles, b0 + boff,
                                                tiv[boff] & 3)
                    w = wcache[boff]
                    cq = civ[q // 16][q % 16] & 3
                    cv = _row_f32(ctiles, (b0 + boff) * _C + c, cq)
                    p = w[0] * cv[0]
                    for k in range(1, 4):
                        p = p + w[k] * cv[k]
                    prods.append(p)
                vecs = prods
                for step in range(4):
                    msk, pidx = masks[step], pidxs[step]
                    vecs = [jnp.where(msk, vecs[2 * i + 1], vecs[2 * i])
                            + _lane_perm(
                                jnp.where(msk, vecs[2 * i], vecs[2 * i + 1]),
                                pidx)
                            for i in range(len(vecs) // 2)]
                out_v[pl.ds((cb + b0) * _C + m * 16, 16)] = (
                    vecs[0] * _INV_SCALE2)
            return carry

        lax.fori_loop(0, _CB // 16, body, 0)

    pltpu.sync_copy(out_v, out_hbm.at[pl.ds(base * _C, _BPW * _C)])


@jax.jit
def _run(tflat, cflat, ttab, ctab):
    mesh = plsc.VectorSubcoreMesh(core_axis_name="c", subcore_axis_name="s")
    call = pl.kernel(
        _asm2vec_body,
        mesh=mesh,
        out_type=jax.ShapeDtypeStruct((_BATCH * _C,), jnp.float32),
        scratch_types=[
            pltpu.VMEM((_BPW,), jnp.int32),
            pltpu.VMEM((_BPW * _C,), jnp.int32),
            pltpu.VMEM((_BPW,), jnp.int32),
            pltpu.VMEM((_BPW * _C,), jnp.int32),
            pltpu.VMEM((_CB, 128), jnp.int32),
            pltpu.VMEM((_CB * _C, 128), jnp.int32),
            pltpu.VMEM((_BPW * _C,), jnp.float32),
            pltpu.SemaphoreType.DMA,
        ],
    )
    return call(tflat, cflat, ttab, ctab).reshape(_BATCH, _C)


def _pack_table(tab):
    q = jnp.clip(jnp.round(tab * _SCALE), -32768.0, 32767.0)
    q16 = q.astype(jnp.int16).reshape(_VOCAB, _EMB // 2, 2)
    packed = jax.lax.bitcast_convert_type(q16, jnp.int32)
    return packed.reshape(_VOCAB // 4, 128)


def kernel(target, context, target_table, context_table):
    tflat = target.reshape(-1).astype(jnp.int32)
    cflat = context.reshape(-1).astype(jnp.int32)
    return _run(tflat, cflat, _pack_table(target_table),
                _pack_table(context_table))


# speedup vs baseline: 2.3137x; 2.3137x over previous
import functools

import jax
import jax.numpy as jnp
from jax import lax
from jax.experimental import pallas as pl
from jax.experimental.pallas import tpu as pltpu
from jax.experimental.pallas import tpu_sc as plsc

_VOCAB = 1000000
_EMB = 64
_BATCH = 16384
_C = 3

_NC = 2
_NS = 16
_NW = _NC * _NS
_BPW = _BATCH // _NW
_CB = 128
_NCHUNK = _BPW // _CB
_SCALE = float(2 ** 17)
_INV_SCALE2 = float(2 ** -34)


def _lane_perm(v, idx):
    return lax.gather(
        v, idx.reshape(16, 1),
        lax.GatherDimensionNumbers(
            offset_dims=(), collapsed_slice_dims=(0,), start_index_map=(0,)),
        slice_sizes=(1,),
        mode=lax.GatherScatterMode.PROMISE_IN_BOUNDS)


def _row_f32(tiles, slot, qoff):
    out = []
    for k in range(2):
        v = tiles[slot, pl.ds(qoff * 32 + 16 * k, 16)]
        lo = (v << 16) >> 16
        hi = v >> 16
        out.append(lo.astype(jnp.float32))
        out.append(hi.astype(jnp.float32))
    return out


def _asm2vec_body(tidx_hbm, cidx_hbm, ttab_hbm, ctab_hbm, out_hbm,
                  tidx_v, cidx_v, tg_v, cg_v, wtiles, ctiles, out_v, sem):
    wid = lax.axis_index("s") * _NC + lax.axis_index("c")
    base = wid * _BPW

    pltpu.sync_copy(tidx_hbm.at[pl.ds(base, _BPW)], tidx_v)
    pltpu.sync_copy(cidx_hbm.at[pl.ds(base * _C, _BPW * _C)], cidx_v)

    def grp_t(j, carry):
        tg_v[pl.ds(16 * j, 16)] = tidx_v[pl.ds(16 * j, 16)] >> 2
        return carry

    def grp_c(j, carry):
        cg_v[pl.ds(16 * j, 16)] = cidx_v[pl.ds(16 * j, 16)] >> 2
        return carry

    lax.fori_loop(0, _BPW // 16, grp_t, 0)
    lax.fori_loop(0, _BPW * _C // 16, grp_c, 0)

    lane = lax.iota(jnp.int32, 16)
    masks = [(lane & sh) != 0 for sh in (1, 2, 4, 8)]
    pidxs = [lane ^ sh for sh in (1, 2, 4, 8)]

    for ck in range(_NCHUNK):
        cb = ck * _CB

        h = [pltpu.async_copy(
            ttab_hbm.at[tg_v.at[pl.ds(cb + 128 * j, 128)]],
            wtiles.at[pl.ds(128 * j, 128)], sem)
            for j in range(_CB // 128)]
        for j in range(_CB * _C // 128):
            h.append(pltpu.async_copy(
                ctab_hbm.at[cg_v.at[pl.ds(cb * _C + 128 * j, 128)]],
                ctiles.at[pl.ds(128 * j, 128)], sem))
        for hh in h:
            hh.wait()

        def body(g, carry):
            b0 = g * 16
            tiv = tidx_v[pl.ds(cb + b0, 16)]
            civ = [cidx_v[pl.ds((cb + b0) * _C + 16 * j, 16)]
                   for j in range(_C)]
            for m in range(_C):
                wcache = {}
                prods = []
                for l in range(16):
                    q = m * 16 + l
                    boff, c = q // _C, q % _C
                    if boff not in wcache:
                        wcache[boff] = _row_f32(wtiles, b0 + boff,
                                                tiv[boff] & 3)
                    w = wcache[boff]
                    cq = civ[q // 16][q % 16] & 3
                    cv = _row_f32(ctiles, (b0 + boff) * _C + c, cq)
                    p = w[0] * cv[0]
                    for k in range(1, 4):
                        p = p + w[k] * cv[k]
                    prods.append(p)
                vecs = prods
                for step in range(4):
                    msk, pidx = masks[step], pidxs[step]
                    vecs = [jnp.where(msk, vecs[2 * i + 1], vecs[2 * i])
                            + _lane_perm(
                                jnp.where(msk, vecs[2 * i], vecs[2 * i + 1]),
                                pidx)
                            for i in range(len(vecs) // 2)]
                out_v[pl.ds((cb + b0) * _C + m * 16, 16)] = (
                    vecs[0] * _INV_SCALE2)
            return carry

        lax.fori_loop(0, _CB // 16, body, 0)

    pltpu.sync_copy(out_v, out_hbm.at[pl.ds(base * _C, _BPW * _C)])


@jax.jit
def _run(tflat, cflat, ttab, ctab):
    mesh = plsc.VectorSubcoreMesh(core_axis_name="c", subcore_axis_name="s")
    call = pl.kernel(
        _asm2vec_body,
        mesh=mesh,
        out_type=jax.ShapeDtypeStruct((_BATCH * _C,), jnp.float32),
        scratch_types=[
            pltpu.VMEM((_BPW,), jnp.int32),
            pltpu.VMEM((_BPW * _C,), jnp.int32),
            pltpu.VMEM((_BPW,), jnp.int32),
            pltpu.VMEM((_BPW * _C,), jnp.int32),
            pltpu.VMEM((_CB, 128), jnp.int32),
            pltpu.VMEM((_CB * _C, 128), jnp.int32),
            pltpu.VMEM((_BPW * _C,), jnp.float32),
            pltpu.SemaphoreType.DMA,
        ],
    )
    return call(tflat, cflat, ttab, ctab).reshape(_BATCH, _C)


def _pack_table(tab):
    q = jnp.clip(jnp.round(tab * _SCALE), -32768.0, 32767.0).astype(jnp.int32)
    packed = (q[:, 32:] << 16) | (q[:, :32] & 0xFFFF)
    return packed.reshape(_VOCAB // 4, 128)


def kernel(target, context, target_table, context_table):
    tflat = target.reshape(-1).astype(jnp.int32)
    cflat = context.reshape(-1).astype(jnp.int32)
    return _run(tflat, cflat, _pack_table(target_table),
                _pack_table(context_table))


# speedup vs baseline: 4.4170x; 1.9091x over previous
import functools

import jax
import jax.numpy as jnp
from jax import lax
from jax.experimental import pallas as pl
from jax.experimental.pallas import tpu as pltpu
from jax.experimental.pallas import tpu_sc as plsc

_VOCAB = 1000000
_EMB = 64
_BATCH = 16384
_C = 3

_NC = 2
_NS = 16
_NW = _NC * _NS
_BPW = _BATCH // _NW
_CB = 128
_NCHUNK = _BPW // _CB
_NSEM = 4


def _lane_perm(v, idx):
    return lax.gather(
        v, idx.reshape(16, 1),
        lax.GatherDimensionNumbers(
            offset_dims=(), collapsed_slice_dims=(0,), start_index_map=(0,)),
        slice_sizes=(1,),
        mode=lax.GatherScatterMode.PROMISE_IN_BOUNDS)


def _asm2vec_body(tidx_hbm, cidx_hbm, ttab_hbm, ctab_hbm, out_hbm,
                  tidx_v, cidx_v, wrows, crows, out_v, *sems):
    wid = lax.axis_index("s") * _NC + lax.axis_index("c")
    base = wid * _BPW

    pltpu.sync_copy(tidx_hbm.at[pl.ds(base, _BPW)], tidx_v)
    pltpu.sync_copy(cidx_hbm.at[pl.ds(base * _C, _BPW * _C)], cidx_v)

    lane = lax.iota(jnp.int32, 16)
    masks = [(lane & sh) != 0 for sh in (1, 2, 4, 8)]
    pidxs = [lane ^ sh for sh in (1, 2, 4, 8)]

    for ck in range(_NCHUNK):
        cb = ck * _CB

        def trow(g, carry):
            iv = tidx_v[pl.ds(cb + g * 16, 16)]
            for k in range(16):
                pltpu.async_copy(ttab_hbm.at[iv[k]], wrows.at[g * 16 + k],
                                 sems[k % _NSEM])
            return carry

        def crow(g, carry):
            iv = cidx_v[pl.ds(cb * _C + g * 16, 16)]
            for k in range(16):
                pltpu.async_copy(ctab_hbm.at[iv[k]], crows.at[g * 16 + k],
                                 sems[k % _NSEM])
            return carry

        lax.fori_loop(0, _CB // 16, trow, 0)
        lax.fori_loop(0, _CB * _C // 16, crow, 0)
        nt = _CB // _NSEM
        nc = _CB * _C // _NSEM
        for s in range(_NSEM):
            pltpu.make_async_copy(
                ttab_hbm.at[pl.ds(0, nt)], wrows.at[pl.ds(0, nt)],
                sems[s]).wait()
            pltpu.make_async_copy(
                ctab_hbm.at[pl.ds(0, nc)], crows.at[pl.ds(0, nc)],
                sems[s]).wait()

        def body(g, carry):
            b0 = g * 16
            for m in range(_C):
                wcache = {}
                prods = []
                for l in range(16):
                    q = m * 16 + l
                    boff, c = q // _C, q % _C
                    if boff not in wcache:
                        wcache[boff] = [wrows[b0 + boff, pl.ds(16 * k, 16)]
                                        for k in range(_EMB // 16)]
                    w = wcache[boff]
                    r = (b0 + boff) * _C + c
                    p = w[0] * crows[r, pl.ds(0, 16)]
                    for k in range(1, _EMB // 16):
                        p = p + w[k] * crows[r, pl.ds(16 * k, 16)]
                    prods.append(p)
                vecs = prods
                for step in range(4):
                    msk, pidx = masks[step], pidxs[step]
                    vecs = [jnp.where(msk, vecs[2 * i + 1], vecs[2 * i])
                            + _lane_perm(
                                jnp.where(msk, vecs[2 * i], vecs[2 * i + 1]),
                                pidx)
                            for i in range(len(vecs) // 2)]
                out_v[pl.ds((cb + b0) * _C + m * 16, 16)] = vecs[0]
            return carry

        lax.fori_loop(0, _CB // 16, body, 0)

    pltpu.sync_copy(out_v, out_hbm.at[pl.ds(base * _C, _BPW * _C)])


@jax.jit
def _run(tflat, cflat, ttab, ctab):
    mesh = plsc.VectorSubcoreMesh(core_axis_name="c", subcore_axis_name="s")
    call = pl.kernel(
        _asm2vec_body,
        mesh=mesh,
        out_type=jax.ShapeDtypeStruct((_BATCH * _C,), jnp.float32),
        scratch_types=[
            pltpu.VMEM((_BPW,), jnp.int32),
            pltpu.VMEM((_BPW * _C,), jnp.int32),
            pltpu.VMEM((_CB, _EMB), jnp.float32),
            pltpu.VMEM((_CB * _C, _EMB), jnp.float32),
            pltpu.VMEM((_BPW * _C,), jnp.float32),
        ] + [pltpu.SemaphoreType.DMA] * _NSEM,
    )
    return call(tflat, cflat, ttab, ctab).reshape(_BATCH, _C)


def kernel(target, context, target_table, context_table):
    tflat = target.reshape(-1).astype(jnp.int32)
    cflat = context.reshape(-1).astype(jnp.int32)
    return _run(tflat, cflat, target_table, context_table)
